# baseline (speedup 1.0000x reference)
import functools

import jax
import jax.numpy as jnp
from jax import lax
from jax.experimental import pallas as pl
from jax.experimental.pallas import tpu as pltpu
from jax.experimental.pallas import tpu_sc as plsc

MAXLEN = 200
EMBED_DIM = 64
BATCH = 1024

NUM_WORKERS = 32
B_PER_W = BATCH // NUM_WORKERS
IDX_SPLIT = 2
IDX_MINOR = MAXLEN // IDX_SPLIT
VECS_PER_ROW = EMBED_DIM // 16


def _sc_body(idx_hbm, tok_hbm, pos_hbm, out_hbm, idx_v, rows_v, pos_v, sem):
    wid = lax.axis_index("s") * 2 + lax.axis_index("c")
    b0 = wid * B_PER_W

    pltpu.sync_copy(pos_hbm, pos_v)

    def do_batch(i, carry):
        b = b0 + i
        pltpu.sync_copy(idx_hbm.at[b], idx_v)
        for c in range(IDX_SPLIT):
            pltpu.async_copy(
                tok_hbm.at[idx_v.at[c]],
                rows_v.at[pl.ds(c * IDX_MINOR, IDX_MINOR)],
                sem,
            ).wait()

        def add_row(r, carry2):
            for j in range(VECS_PER_ROW):
                sl = pl.ds(j * 16, 16)
                rows_v[r, sl] = rows_v[r, sl] + pos_v[r, sl]
            return carry2

        lax.fori_loop(0, MAXLEN, add_row, 0)
        pltpu.sync_copy(rows_v, out_hbm.at[b])
        return carry

    lax.fori_loop(0, B_PER_W, do_batch, 0)


def kernel(inputs, token_table, pos_table):
    idx3 = inputs.reshape(BATCH, IDX_SPLIT, IDX_MINOR).astype(jnp.int32)
    mesh = plsc.VectorSubcoreMesh(core_axis_name="c", subcore_axis_name="s")
    k = functools.partial(
        pl.kernel,
        out_type=jax.ShapeDtypeStruct((BATCH, MAXLEN, EMBED_DIM), jnp.float32),
        mesh=mesh,
        scratch_types=[
            pltpu.VMEM((IDX_SPLIT, IDX_MINOR), jnp.int32),
            pltpu.VMEM((MAXLEN, EMBED_DIM), jnp.float32),
            pltpu.VMEM((MAXLEN, EMBED_DIM), jnp.float32),
            pltpu.SemaphoreType.DMA,
        ],
        compiler_params=pltpu.CompilerParams(use_tc_tiling_on_sc=False),
    )(_sc_body)
    return k(idx3, token_table, pos_table)


# speedup vs baseline: 1.1060x; 1.1060x over previous
import functools

import jax
import jax.numpy as jnp
from jax import lax
from jax.experimental import pallas as pl
from jax.experimental.pallas import tpu as pltpu
from jax.experimental.pallas import tpu_sc as plsc

MAXLEN = 200
EMBED_DIM = 64
BATCH = 1024

NUM_WORKERS = 32
B_PER_W = BATCH // NUM_WORKERS
IDX_SPLIT = 2
IDX_MINOR = MAXLEN // IDX_SPLIT
VECS_PER_ROW = EMBED_DIM // 16
NBUF = 4
GATHER_AHEAD = 2


def _sc_body(idx_hbm, tok_hbm, pos_hbm, out_hbm, idx_v, rows_v, pos_v,
             sem_g, sem_o):
    wid = lax.axis_index("s") * 2 + lax.axis_index("c")
    b0 = wid * B_PER_W

    pltpu.sync_copy(pos_hbm, pos_v)
    pltpu.sync_copy(idx_hbm.at[pl.ds(b0, B_PER_W)], idx_v)

    def start_gather(i, b):
        for c in range(IDX_SPLIT):
            pltpu.make_async_copy(
                tok_hbm.at[idx_v.at[i, c]],
                rows_v.at[b, pl.ds(c * IDX_MINOR, IDX_MINOR)],
                sem_g,
            ).start()

    def wait_gather(i, b):
        for c in range(IDX_SPLIT):
            pltpu.make_async_copy(
                tok_hbm.at[idx_v.at[i, c]],
                rows_v.at[b, pl.ds(c * IDX_MINOR, IDX_MINOR)],
                sem_g,
            ).wait()

    def out_copy(i, b):
        return pltpu.make_async_copy(rows_v.at[b], out_hbm.at[b0 + i], sem_o)

    for i in range(GATHER_AHEAD):
        start_gather(i, i % NBUF)

    def outer(io, carry):
        for b_off in range(NBUF):
            i = io * NBUF + b_off
            b = b_off
            @pl.when(i >= NBUF - GATHER_AHEAD)
            def _():
                out_copy(i - (NBUF - GATHER_AHEAD),
                         (i + GATHER_AHEAD) % NBUF).wait()

            @pl.when(i + GATHER_AHEAD < B_PER_W)
            def _():
                start_gather(i + GATHER_AHEAD, (i + GATHER_AHEAD) % NBUF)

            wait_gather(i, b)

            @plsc.parallel_loop(0, MAXLEN, 1, unroll=4)
            def add_row(r):
                for j in range(VECS_PER_ROW):
                    sl = pl.ds(j * 16, 16)
                    rows_v[b, r, sl] = rows_v[b, r, sl] + pos_v[r, sl]

            out_copy(i, b).start()
        return carry

    lax.fori_loop(0, B_PER_W // NBUF, outer, 0)

    for i in range(B_PER_W - (NBUF - GATHER_AHEAD), B_PER_W):
        out_copy(i, i % NBUF).wait()


def kernel(inputs, token_table, pos_table):
    idx3 = inputs.reshape(BATCH, IDX_SPLIT, IDX_MINOR).astype(jnp.int32)
    mesh = plsc.VectorSubcoreMesh(core_axis_name="c", subcore_axis_name="s")
    k = functools.partial(
        pl.kernel,
        out_type=jax.ShapeDtypeStruct((BATCH, MAXLEN, EMBED_DIM), jnp.float32),
        mesh=mesh,
        scratch_types=[
            pltpu.VMEM((B_PER_W, IDX_SPLIT, IDX_MINOR), jnp.int32),
            pltpu.VMEM((NBUF, MAXLEN, EMBED_DIM), jnp.float32),
            pltpu.VMEM((MAXLEN, EMBED_DIM), jnp.float32),
            pltpu.SemaphoreType.DMA,
            pltpu.SemaphoreType.DMA,
        ],
        compiler_params=pltpu.CompilerParams(use_tc_tiling_on_sc=False),
    )(_sc_body)
    return k(idx3, token_table, pos_table)
